# baseline (speedup 1.0000x reference)
import jax
import jax.numpy as jnp
from jax import lax
from jax.experimental import pallas as pl
from jax.experimental.pallas import tpu as pltpu
from jax.experimental.pallas import tpu_sc as plsc

_FOUR_MASKS = 8
_FOUR_STATES = 65536
_FIVE_MASKS = 8
_FIVE_STATES = 59049
_D = 64
_H = 32
_B = 16384
_M = 16

_NC, _NS, _L = 2, 16, 16
_NW = _NC * _NS
_BPW = _B // _NW
_CHUNK = 32
_NCHUNK = _BPW // _CHUNK
_CROWS = _CHUNK * _M
_GSPLIT = 4
_GROWS = _CROWS // _GSPLIT


def _sc_pool_body(idx_hbm, table_hbm, acc_hbm, idx_v, rows_v, out_v, sem0, sem1):
    wid = lax.axis_index("s") * _NC + lax.axis_index("c")
    ebase = wid * _BPW
    sems = (sem0, sem1)

    lane = lax.iota(jnp.int32, 16)
    offs = jnp.where(
        lane < _FOUR_MASKS,
        lane * _FOUR_STATES,
        _FOUR_MASKS * _FOUR_STATES + (lane - _FOUR_MASKS) * _FIVE_STATES,
    )

    def fire(g, buf):
        pltpu.sync_copy(
            idx_hbm.at[pl.ds((ebase + g * _CHUNK) * _M, _CROWS)],
            idx_v.at[buf],
        )

        def fix(e, _):
            sl = pl.ds(e * _M, _M)
            idx_v[buf, sl] = idx_v[buf, sl] + offs
            return 0

        lax.fori_loop(0, _CHUNK, fix, 0)
        return [
            pltpu.async_copy(
                table_hbm.at[idx_v.at[buf, pl.ds(j * _GROWS, _GROWS)]],
                rows_v.at[buf, pl.ds(j * _GROWS, _GROWS)],
                sems[buf],
            )
            for j in range(_GSPLIT)
        ]

    def pool(g, buf):
        def elem(e, _):
            row0 = e * _M
            for q in range(_D // _L):
                cs = pl.ds(q * _L, _L)
                s = rows_v[buf, row0, cs]
                for r in range(1, _M):
                    s = s + rows_v[buf, row0 + r, cs]
                out_v[buf, e, cs] = jnp.minimum(jnp.maximum(s, 0.0), 1.0)
            return 0

        lax.fori_loop(0, _CHUNK, elem, 0)
        pltpu.sync_copy(
            out_v.at[buf],
            acc_hbm.at[pl.ds(ebase + g * _CHUNK, _CHUNK)],
        )

    pending = fire(0, 0)
    for g in range(_NCHUNK):
        buf = g & 1
        current = pending
        if g + 1 < _NCHUNK:
            pending = fire(g + 1, 1 - buf)
        for h in current:
            h.wait()
        pool(g, buf)


_sc_pool = pl.kernel(
    _sc_pool_body,
    out_type=jax.ShapeDtypeStruct((_B, _D), jnp.float32),
    mesh=plsc.VectorSubcoreMesh(
        core_axis_name="c", subcore_axis_name="s",
        num_cores=_NC, num_subcores=_NS,
    ),
    scratch_types=[
        pltpu.VMEM((2, _CROWS), jnp.int32),
        pltpu.VMEM((2, _CROWS, _D), jnp.float32),
        pltpu.VMEM((2, _CHUNK, _D), jnp.float32),
        pltpu.SemaphoreType.DMA,
        pltpu.SemaphoreType.DMA,
    ],
    compiler_params=pltpu.CompilerParams(use_tc_tiling_on_sc=False),
)


_MLP_BLK = 2048


def _mlp_body(acc_ref, w1_ref, b1_ref, w2_ref, b2_ref, out_ref):
    a = acc_ref[...]
    h = jnp.dot(a, w1_ref[...], preferred_element_type=jnp.float32) + b1_ref[...]
    h = jnp.minimum(jnp.maximum(h, 0.0), 1.0)
    o = jnp.sum(h * w2_ref[...], axis=1) + b2_ref[0, 0]
    out_ref[...] = jnp.tanh(o)


_mlp = pl.pallas_call(
    _mlp_body,
    grid=(_B // _MLP_BLK,),
    in_specs=[
        pl.BlockSpec((_MLP_BLK, _D), lambda i: (i, 0)),
        pl.BlockSpec((_D, _H), lambda i: (0, 0)),
        pl.BlockSpec((1, _H), lambda i: (0, 0)),
        pl.BlockSpec((1, _H), lambda i: (0, 0)),
        pl.BlockSpec(memory_space=pltpu.SMEM),
    ],
    out_specs=pl.BlockSpec((_MLP_BLK,), lambda i: (i,)),
    out_shape=jax.ShapeDtypeStruct((_B,), jnp.float32),
)


def kernel(four_states, five_states, table, hidden_w, hidden_b, output_w, output_b):
    idx = jnp.concatenate([four_states, five_states], axis=1).reshape(-1)
    acc = _sc_pool(idx, table)
    return _mlp(
        acc,
        hidden_w,
        hidden_b.reshape(1, _H),
        output_w.reshape(1, _H),
        output_b.reshape(1, 1),
    )
